# baseline (speedup 1.0000x reference)
import functools

import jax
import jax.numpy as jnp
from jax import lax
from jax.experimental import pallas as pl
from jax.experimental.pallas import tpu as pltpu
from jax.experimental.pallas import tpu_sc as plsc

B, P = 4, 256
C1, C2 = P // 2, P // 8
R1, R2 = 0.2, 0.4
NEG = -1e9
THRESH = -5e8


def _fps_level(px, py, pz, n, cidx_n):
    bsz, npts = px.shape
    jidx = jax.lax.broadcasted_iota(jnp.int32, (bsz, npts), 1)
    qx0 = px[:, 0:1]
    qy0 = py[:, 0:1]
    qz0 = pz[:, 0:1]
    dx = px - qx0
    dy = py - qy0
    dz = pz - qz0
    dist = dx * dx + dy * dy + dz * dz
    col0 = cidx_n == 0
    self_f = jnp.zeros((bsz, n), jnp.float32)
    cx = jnp.where(col0, qx0, 0.0)
    cy = jnp.where(col0, qy0, 0.0)
    cz = jnp.where(col0, qz0, 0.0)

    def body(i, state):
        self_f, cx, cy, cz, dist = state
        m = jnp.max(dist, axis=1, keepdims=True)
        eq = dist == m
        nxt = jnp.min(jnp.where(eq, jidx, npts), axis=1, keepdims=True)
        pick = jidx == nxt
        qx = jnp.max(jnp.where(pick, px, -jnp.inf), axis=1, keepdims=True)
        qy = jnp.max(jnp.where(pick, py, -jnp.inf), axis=1, keepdims=True)
        qz = jnp.max(jnp.where(pick, pz, -jnp.inf), axis=1, keepdims=True)
        dx = px - qx
        dy = py - qy
        dz = pz - qz
        d2 = dx * dx + dy * dy + dz * dz
        dist = jnp.minimum(dist, d2)
        col = cidx_n == i
        nxtf = nxt.astype(jnp.float32)
        self_f = jnp.where(col, nxtf, self_f)
        cx = jnp.where(col, qx, cx)
        cy = jnp.where(col, qy, cy)
        cz = jnp.where(col, qz, cz)
        return (self_f, cx, cy, cz, dist)

    self_f, cx, cy, cz, _ = jax.lax.fori_loop(
        1, n, body, (self_f, cx, cy, cz, dist))
    return self_f, cx, cy, cz


_DNUMS = lax.GatherDimensionNumbers(
    offset_dims=(), collapsed_slice_dims=(0,), start_index_map=(0,))


def _take16(v, lanev):
    return lax.gather(v, lanev[:, None], _DNUMS, (1,),
                      mode=lax.GatherScatterMode.PROMISE_IN_BOUNDS)


def _allmax(v, iota):
    for s in (8, 4, 2, 1):
        v = jnp.maximum(v, _take16(v, iota ^ s))
    return v


def _allmin_i32(v, iota):
    for s in (8, 4, 2, 1):
        v = jnp.minimum(v, _take16(v, iota ^ s))
    return v


def _lane_write(ref, i, lanev, iota, val_splat):
    base = pl.multiple_of((i // 16) * 16, 16)
    old = ref[pl.ds(base, 16)]
    ref[pl.ds(base, 16)] = jnp.where(iota == lanev, val_splat, old)


def _fps_sc_level(pts, npts, n, dist_ref, sel_ref, ox, oy, oz):
    nch = npts // 16
    iota = lax.iota(jnp.int32, 16)
    q0x = _take16(pts[0][pl.ds(0, 16)], jnp.zeros((16,), jnp.int32))
    q0y = _take16(pts[1][pl.ds(0, 16)], jnp.zeros((16,), jnp.int32))
    q0z = _take16(pts[2][pl.ds(0, 16)], jnp.zeros((16,), jnp.int32))
    for k in range(nch):
        dx = pts[0][pl.ds(16 * k, 16)] - q0x
        dy = pts[1][pl.ds(16 * k, 16)] - q0y
        dz = pts[2][pl.ds(16 * k, 16)] - q0z
        dist_ref[pl.ds(16 * k, 16)] = dx * dx + dy * dy + dz * dz
    sel_ref[pl.ds(0, 16)] = jnp.zeros((16,), jnp.float32)
    ox[pl.ds(0, 16)] = jnp.where(iota == 0, q0x, 0.0)
    oy[pl.ds(0, 16)] = jnp.where(iota == 0, q0y, 0.0)
    oz[pl.ds(0, 16)] = jnp.where(iota == 0, q0z, 0.0)

    def body(i, iv):
        acc = dist_ref[pl.ds(0, 16)]
        for k in range(1, nch):
            acc = jnp.maximum(acc, dist_ref[pl.ds(16 * k, 16)])
        m = _allmax(acc, iota)
        accid = jnp.full((16,), npts, jnp.int32)
        for k in range(nch):
            v = dist_ref[pl.ds(16 * k, 16)]
            cand = jnp.where(v == m, iota + (16 * k), npts)
            accid = jnp.minimum(accid, cand)
        nxtv = _allmin_i32(accid, iota)
        nxt_s = nxtv[0]
        base = pl.multiple_of((nxt_s // 16) * 16, 16)
        lanev = nxtv & 15
        qx = _take16(pts[0][pl.ds(base, 16)], lanev)
        qy = _take16(pts[1][pl.ds(base, 16)], lanev)
        qz = _take16(pts[2][pl.ds(base, 16)], lanev)
        for k in range(nch):
            dx = pts[0][pl.ds(16 * k, 16)] - qx
            dy = pts[1][pl.ds(16 * k, 16)] - qy
            dz = pts[2][pl.ds(16 * k, 16)] - qz
            d2 = dx * dx + dy * dy + dz * dz
            dist_ref[pl.ds(16 * k, 16)] = jnp.minimum(
                dist_ref[pl.ds(16 * k, 16)], d2)
        ilane = iv & 15
        _lane_write(sel_ref, i, ilane, iota, nxtv.astype(jnp.float32))
        _lane_write(ox, i, ilane, iota, qx)
        _lane_write(oy, i, ilane, iota, qy)
        _lane_write(oz, i, ilane, iota, qz)
        return iv + 1

    lax.fori_loop(1, n, body, jnp.ones((16,), jnp.int32))


def _fps_sc(posTb):
    mesh = plsc.VectorSubcoreMesh(core_axis_name="c", subcore_axis_name="s")
    f32 = jnp.float32

    @functools.partial(
        pl.kernel,
        mesh=mesh,
        out_type=[
            jax.ShapeDtypeStruct((B * C1,), f32),
            jax.ShapeDtypeStruct((B * C1,), f32),
            jax.ShapeDtypeStruct((B * C1,), f32),
            jax.ShapeDtypeStruct((B * C1,), f32),
            jax.ShapeDtypeStruct((B * C2,), f32),
            jax.ShapeDtypeStruct((B * C2,), f32),
            jax.ShapeDtypeStruct((B * C2,), f32),
            jax.ShapeDtypeStruct((B * C2,), f32),
        ],
        scratch_types=[
            pltpu.VMEM((P,), f32),
            pltpu.VMEM((P,), f32),
            pltpu.VMEM((P,), f32),
            pltpu.VMEM((P,), f32),
            pltpu.VMEM((C1,), f32),
            pltpu.VMEM((C1,), f32),
            pltpu.VMEM((C1,), f32),
            pltpu.VMEM((C1,), f32),
            pltpu.VMEM((C1,), f32),
            pltpu.VMEM((C2,), f32),
            pltpu.VMEM((C2,), f32),
            pltpu.VMEM((C2,), f32),
            pltpu.VMEM((C2,), f32),
        ],
    )
    def fps_kernel(pos_hbm, sel1_o, cx1_o, cy1_o, cz1_o, sel2_o, c2x_o,
                   c2y_o, c2z_o, px, py, pz, dist1, sel1, cx, cy, cz, dist2,
                   sel2, c2x, c2y, c2z):
        wid = lax.axis_index("s") * 2 + lax.axis_index("c")

        @pl.when(wid < B)
        def _():
            pltpu.sync_copy(pos_hbm.at[pl.ds(wid * (3 * P), P)], px)
            pltpu.sync_copy(pos_hbm.at[pl.ds(wid * (3 * P) + P, P)], py)
            pltpu.sync_copy(pos_hbm.at[pl.ds(wid * (3 * P) + 2 * P, P)], pz)
            _fps_sc_level([px, py, pz], P, C1, dist1, sel1, cx, cy, cz)
            _fps_sc_level([cx, cy, cz], C1, C2, dist2, sel2, c2x, c2y, c2z)

            pltpu.sync_copy(sel1, sel1_o.at[pl.ds(wid * C1, C1)])
            pltpu.sync_copy(cx, cx1_o.at[pl.ds(wid * C1, C1)])
            pltpu.sync_copy(cy, cy1_o.at[pl.ds(wid * C1, C1)])
            pltpu.sync_copy(cz, cz1_o.at[pl.ds(wid * C1, C1)])
            pltpu.sync_copy(sel2, sel2_o.at[pl.ds(wid * C2, C2)])
            pltpu.sync_copy(c2x, c2x_o.at[pl.ds(wid * C2, C2)])
            pltpu.sync_copy(c2y, c2y_o.at[pl.ds(wid * C2, C2)])
            pltpu.sync_copy(c2z, c2z_o.at[pl.ds(wid * C2, C2)])

    outs = fps_kernel(posTb.reshape(-1))
    sel1, cx1, cy1, cz1 = (o.reshape(B, C1) for o in outs[:4])
    sel2, cx2, cy2, cz2 = (o.reshape(B, C2) for o in outs[4:])
    return sel1, cx1, cy1, cz1, sel2, cx2, cy2, cz2


def _fps_body(posT_ref, sel1_ref, cen1T_ref, sel2_ref, cen2T_ref):
    px = posT_ref[0]
    py = posT_ref[1]
    pz = posT_ref[2]
    cidx1 = jax.lax.broadcasted_iota(jnp.int32, (B, C1), 1)
    s1, cx1, cy1, cz1 = _fps_level(px, py, pz, C1, cidx1)
    sel1_ref[...] = s1
    cen1T_ref[0] = cx1
    cen1T_ref[1] = cy1
    cen1T_ref[2] = cz1
    cidx2 = jax.lax.broadcasted_iota(jnp.int32, (B, C2), 1)
    s2, cx2, cy2, cz2 = _fps_level(cx1, cy1, cz1, C2, cidx2)
    sel2_ref[...] = s2
    cen2T_ref[0] = cx2
    cen2T_ref[1] = cy2
    cen2T_ref[2] = cz2


def _stage1_body(pos_ref, posT_ref, cen1_ref, w1_ref, b1_ref, w2a_ref, b2_ref,
                 x1_ref):
    pos2 = pos_ref[0]
    posT2 = posT_ref[0]
    cen4 = cen1_ref[0]
    cen3 = cen4[:, 0:3]
    w1 = w1_ref[...]
    a = jnp.dot(pos2, w1, preferred_element_type=jnp.float32)
    d = jnp.dot(cen3, w1, preferred_element_type=jnp.float32)
    pre = (a + b1_ref[...])[None, :, :] - d[:, None, :]
    h = jax.nn.relu(pre)

    dx = posT2[0:1, :] - cen4[:, 0:1]
    dy = posT2[1:2, :] - cen4[:, 1:2]
    dz = posT2[2:3, :] - cen4[:, 2:3]
    sq = dx * dx + dy * dy + dz * dz
    jidx = jax.lax.broadcasted_iota(jnp.int32, (C1, P), 1)
    mask = (sq < R1 * R1) & (jidx != cen4[:, 3:4].astype(jnp.int32))
    pen = jnp.where(mask, 0.0, NEG).astype(jnp.bfloat16)[:, :, None]
    msg = jnp.dot(h.reshape(C1 * P, 64).astype(jnp.bfloat16),
                  w2a_ref[...].astype(jnp.bfloat16),
                  preferred_element_type=jnp.float32).astype(jnp.bfloat16)
    msgm = msg.reshape(C1, P, 128) + pen
    m1 = jnp.max(msgm, axis=1)
    b2_16 = b2_ref[...].astype(jnp.bfloat16)
    x1_ref[0] = jnp.where(m1 > THRESH, m1 + b2_16,
                          jnp.bfloat16(0.0))


def _stage2_body(x1_ref, cen1_ref, cen1T_ref, cen2_ref, w21a_ref, w21b_ref,
                 b21_ref, w22a_ref, b22_ref, out_ref):
    x1 = x1_ref[0]
    cen4 = cen1_ref[0]
    cen1T2 = cen1T_ref[0]
    cen3 = cen4[:, 0:3]
    cen24 = cen2_ref[0]
    w21b = w21b_ref[...]
    u = (jnp.dot(x1, w21a_ref[...].astype(jnp.bfloat16),
                 preferred_element_type=jnp.float32)
         + jnp.dot(cen3, w21b, preferred_element_type=jnp.float32)
         + b21_ref[...])
    v = jnp.dot(cen24[:, 0:3], w21b, preferred_element_type=jnp.float32)
    u16 = u.astype(jnp.bfloat16)
    v16 = v.astype(jnp.bfloat16)
    pre2 = u16[None, :, :] - v16[:, None, :]
    h2 = jax.nn.relu(pre2)

    dx2 = cen1T2[0:1, :] - cen24[:, 0:1]
    dy2 = cen1T2[1:2, :] - cen24[:, 1:2]
    dz2 = cen1T2[2:3, :] - cen24[:, 2:3]
    sq2 = dx2 * dx2 + dy2 * dy2 + dz2 * dz2
    jidx2 = jax.lax.broadcasted_iota(jnp.int32, (C2, C1), 1)
    mask2 = (sq2 < R2 * R2) & (jidx2 != cen24[:, 3:4].astype(jnp.int32))
    pen2 = jnp.where(mask2, 0.0, NEG).astype(jnp.bfloat16)[:, :, None]
    msg2 = jnp.dot(h2.reshape(C2 * C1, 256), w22a_ref[...].astype(jnp.bfloat16),
                   preferred_element_type=jnp.float32).astype(jnp.bfloat16)
    msgm2 = msg2.reshape(C2, C1, 512) + pen2
    m2 = jnp.max(msgm2, axis=1).astype(jnp.float32)
    x2 = jnp.where(m2 > THRESH, m2 + b22_ref[...], 0.0)
    out_ref[0, 0] = jnp.max(x2, axis=0)


def kernel(pos, batch, sa1_W1, sa1_b1, sa1_W2, sa1_b2, sa2_W1, sa2_b1,
           sa2_W2, sa2_b2):
    del batch
    pos_b = pos.reshape(B, P, 3)
    posTb = jnp.transpose(pos_b, (0, 2, 1))

    sel1, cx1, cy1, cz1, sel2, cx2, cy2, cz2 = _fps_sc(posTb)
    cen1aug = jnp.stack([cx1, cy1, cz1, sel1], axis=-1)
    cen2aug = jnp.stack([cx2, cy2, cz2, sel2], axis=-1)

    x1 = pl.pallas_call(
        _stage1_body,
        grid=(B,),
        in_specs=[
            pl.BlockSpec((1, P, 3), lambda b: (b, 0, 0)),
            pl.BlockSpec((1, 3, P), lambda b: (b, 0, 0)),
            pl.BlockSpec((1, C1, 4), lambda b: (b, 0, 0)),
            pl.BlockSpec((3, 64), lambda b: (0, 0)),
            pl.BlockSpec((1, 64), lambda b: (0, 0)),
            pl.BlockSpec((64, 128), lambda b: (0, 0)),
            pl.BlockSpec((1, 128), lambda b: (0, 0)),
        ],
        out_specs=pl.BlockSpec((1, C1, 128), lambda b: (b, 0, 0)),
        out_shape=jax.ShapeDtypeStruct((B, C1, 128), jnp.bfloat16),
    )(pos_b, posTb, cen1aug, sa1_W1, sa1_b1.reshape(1, 64), sa1_W2,
      sa1_b2.reshape(1, 128))

    out = pl.pallas_call(
        _stage2_body,
        grid=(B,),
        in_specs=[
            pl.BlockSpec((1, C1, 128), lambda b: (b, 0, 0)),
            pl.BlockSpec((1, C1, 4), lambda b: (b, 0, 0)),
            pl.BlockSpec((1, 3, C1), lambda b: (b, 0, 0)),
            pl.BlockSpec((1, C2, 4), lambda b: (b, 0, 0)),
            pl.BlockSpec((128, 256), lambda b: (0, 0)),
            pl.BlockSpec((3, 256), lambda b: (0, 0)),
            pl.BlockSpec((1, 256), lambda b: (0, 0)),
            pl.BlockSpec((256, 512), lambda b: (0, 0)),
            pl.BlockSpec((1, 512), lambda b: (0, 0)),
        ],
        out_specs=pl.BlockSpec((1, 1, 512), lambda b: (b, 0, 0)),
        out_shape=jax.ShapeDtypeStruct((B, 1, 512), jnp.float32),
    )(x1, cen1aug, jnp.stack([cx1, cy1, cz1], axis=1), cen2aug,
      sa2_W1[:128], sa2_W1[128:], sa2_b1.reshape(1, 256), sa2_W2,
      sa2_b2.reshape(1, 512))

    return out.reshape(B, 512)


# speedup vs baseline: 1.0340x; 1.0340x over previous
import functools

import jax
import jax.numpy as jnp
from jax import lax
from jax.experimental import pallas as pl
from jax.experimental.pallas import tpu as pltpu
from jax.experimental.pallas import tpu_sc as plsc

B, P = 4, 256
C1, C2 = P // 2, P // 8
R1, R2 = 0.2, 0.4
NEG = -1e9
THRESH = -5e8


def _fps_level(px, py, pz, n, cidx_n):
    bsz, npts = px.shape
    jidx = jax.lax.broadcasted_iota(jnp.int32, (bsz, npts), 1)
    qx0 = px[:, 0:1]
    qy0 = py[:, 0:1]
    qz0 = pz[:, 0:1]
    dx = px - qx0
    dy = py - qy0
    dz = pz - qz0
    dist = dx * dx + dy * dy + dz * dz
    col0 = cidx_n == 0
    self_f = jnp.zeros((bsz, n), jnp.float32)
    cx = jnp.where(col0, qx0, 0.0)
    cy = jnp.where(col0, qy0, 0.0)
    cz = jnp.where(col0, qz0, 0.0)

    def body(i, state):
        self_f, cx, cy, cz, dist = state
        m = jnp.max(dist, axis=1, keepdims=True)
        eq = dist == m
        nxt = jnp.min(jnp.where(eq, jidx, npts), axis=1, keepdims=True)
        pick = jidx == nxt
        qx = jnp.max(jnp.where(pick, px, -jnp.inf), axis=1, keepdims=True)
        qy = jnp.max(jnp.where(pick, py, -jnp.inf), axis=1, keepdims=True)
        qz = jnp.max(jnp.where(pick, pz, -jnp.inf), axis=1, keepdims=True)
        dx = px - qx
        dy = py - qy
        dz = pz - qz
        d2 = dx * dx + dy * dy + dz * dz
        dist = jnp.minimum(dist, d2)
        col = cidx_n == i
        nxtf = nxt.astype(jnp.float32)
        self_f = jnp.where(col, nxtf, self_f)
        cx = jnp.where(col, qx, cx)
        cy = jnp.where(col, qy, cy)
        cz = jnp.where(col, qz, cz)
        return (self_f, cx, cy, cz, dist)

    self_f, cx, cy, cz, _ = jax.lax.fori_loop(
        1, n, body, (self_f, cx, cy, cz, dist))
    return self_f, cx, cy, cz


_DNUMS = lax.GatherDimensionNumbers(
    offset_dims=(), collapsed_slice_dims=(0,), start_index_map=(0,))


def _take16(v, lanev):
    return lax.gather(v, lanev[:, None], _DNUMS, (1,),
                      mode=lax.GatherScatterMode.PROMISE_IN_BOUNDS)


def _allmax(v, iota):
    for s in (8, 4, 2, 1):
        v = jnp.maximum(v, _take16(v, iota ^ s))
    return v


def _allmin_i32(v, iota):
    for s in (8, 4, 2, 1):
        v = jnp.minimum(v, _take16(v, iota ^ s))
    return v


def _lane_write(ref, i, lanev, iota, val_splat):
    base = pl.multiple_of((i // 16) * 16, 16)
    old = ref[pl.ds(base, 16)]
    ref[pl.ds(base, 16)] = jnp.where(iota == lanev, val_splat, old)


def _fps_sc_level(pts, npts, n, sel_ref, ox, oy, oz):
    nch = npts // 16
    iota = lax.iota(jnp.int32, 16)
    q0x = _take16(pts[0][pl.ds(0, 16)], jnp.zeros((16,), jnp.int32))
    q0y = _take16(pts[1][pl.ds(0, 16)], jnp.zeros((16,), jnp.int32))
    q0z = _take16(pts[2][pl.ds(0, 16)], jnp.zeros((16,), jnp.int32))
    dist0 = []
    acc0 = None
    for k in range(nch):
        dx = pts[0][pl.ds(16 * k, 16)] - q0x
        dy = pts[1][pl.ds(16 * k, 16)] - q0y
        dz = pts[2][pl.ds(16 * k, 16)] - q0z
        d = dx * dx + dy * dy + dz * dz
        dist0.append(d)
        acc0 = d if acc0 is None else jnp.maximum(acc0, d)
    sel_ref[pl.ds(0, 16)] = jnp.zeros((16,), jnp.float32)
    ox[pl.ds(0, 16)] = jnp.where(iota == 0, q0x, 0.0)
    oy[pl.ds(0, 16)] = jnp.where(iota == 0, q0y, 0.0)
    oz[pl.ds(0, 16)] = jnp.where(iota == 0, q0z, 0.0)

    def body(i, carry):
        iv, acc = carry[0], carry[1]
        dist = carry[2:]
        m = _allmax(acc, iota)
        accid = jnp.full((16,), npts, jnp.int32)
        for k in range(nch):
            cand = jnp.where(dist[k] == m, iota + (16 * k), npts)
            accid = jnp.minimum(accid, cand)
        nxtv = _allmin_i32(accid, iota)
        nxt_s = nxtv[0]
        base = pl.multiple_of((nxt_s // 16) * 16, 16)
        lanev = nxtv & 15
        qx = _take16(pts[0][pl.ds(base, 16)], lanev)
        qy = _take16(pts[1][pl.ds(base, 16)], lanev)
        qz = _take16(pts[2][pl.ds(base, 16)], lanev)
        newdist = []
        newacc = None
        for k in range(nch):
            dx = pts[0][pl.ds(16 * k, 16)] - qx
            dy = pts[1][pl.ds(16 * k, 16)] - qy
            dz = pts[2][pl.ds(16 * k, 16)] - qz
            d2 = dx * dx + dy * dy + dz * dz
            nd = jnp.minimum(dist[k], d2)
            newdist.append(nd)
            newacc = nd if newacc is None else jnp.maximum(newacc, nd)
        ilane = iv & 15
        _lane_write(sel_ref, i, ilane, iota, nxtv.astype(jnp.float32))
        _lane_write(ox, i, ilane, iota, qx)
        _lane_write(oy, i, ilane, iota, qy)
        _lane_write(oz, i, ilane, iota, qz)
        return (iv + 1, newacc) + tuple(newdist)

    lax.fori_loop(1, n, body,
                  (jnp.ones((16,), jnp.int32), acc0) + tuple(dist0))


def _fps_sc(posTb):
    mesh = plsc.VectorSubcoreMesh(core_axis_name="c", subcore_axis_name="s")
    f32 = jnp.float32

    @functools.partial(
        pl.kernel,
        mesh=mesh,
        out_type=[
            jax.ShapeDtypeStruct((B * C1,), f32),
            jax.ShapeDtypeStruct((B * C1,), f32),
            jax.ShapeDtypeStruct((B * C1,), f32),
            jax.ShapeDtypeStruct((B * C1,), f32),
            jax.ShapeDtypeStruct((B * C2,), f32),
            jax.ShapeDtypeStruct((B * C2,), f32),
            jax.ShapeDtypeStruct((B * C2,), f32),
            jax.ShapeDtypeStruct((B * C2,), f32),
        ],
        scratch_types=[
            pltpu.VMEM((P,), f32),
            pltpu.VMEM((P,), f32),
            pltpu.VMEM((P,), f32),
            pltpu.VMEM((C1,), f32),
            pltpu.VMEM((C1,), f32),
            pltpu.VMEM((C1,), f32),
            pltpu.VMEM((C1,), f32),
            pltpu.VMEM((C2,), f32),
            pltpu.VMEM((C2,), f32),
            pltpu.VMEM((C2,), f32),
            pltpu.VMEM((C2,), f32),
        ],
    )
    def fps_kernel(pos_hbm, sel1_o, cx1_o, cy1_o, cz1_o, sel2_o, c2x_o,
                   c2y_o, c2z_o, px, py, pz, sel1, cx, cy, cz,
                   sel2, c2x, c2y, c2z):
        wid = lax.axis_index("s") * 2 + lax.axis_index("c")

        @pl.when(wid < B)
        def _():
            pltpu.sync_copy(pos_hbm.at[pl.ds(wid * (3 * P), P)], px)
            pltpu.sync_copy(pos_hbm.at[pl.ds(wid * (3 * P) + P, P)], py)
            pltpu.sync_copy(pos_hbm.at[pl.ds(wid * (3 * P) + 2 * P, P)], pz)
            _fps_sc_level([px, py, pz], P, C1, sel1, cx, cy, cz)
            _fps_sc_level([cx, cy, cz], C1, C2, sel2, c2x, c2y, c2z)

            pltpu.sync_copy(sel1, sel1_o.at[pl.ds(wid * C1, C1)])
            pltpu.sync_copy(cx, cx1_o.at[pl.ds(wid * C1, C1)])
            pltpu.sync_copy(cy, cy1_o.at[pl.ds(wid * C1, C1)])
            pltpu.sync_copy(cz, cz1_o.at[pl.ds(wid * C1, C1)])
            pltpu.sync_copy(sel2, sel2_o.at[pl.ds(wid * C2, C2)])
            pltpu.sync_copy(c2x, c2x_o.at[pl.ds(wid * C2, C2)])
            pltpu.sync_copy(c2y, c2y_o.at[pl.ds(wid * C2, C2)])
            pltpu.sync_copy(c2z, c2z_o.at[pl.ds(wid * C2, C2)])

    outs = fps_kernel(posTb.reshape(-1))
    sel1, cx1, cy1, cz1 = (o.reshape(B, C1) for o in outs[:4])
    sel2, cx2, cy2, cz2 = (o.reshape(B, C2) for o in outs[4:])
    return sel1, cx1, cy1, cz1, sel2, cx2, cy2, cz2


def _fps_body(posT_ref, sel1_ref, cen1T_ref, sel2_ref, cen2T_ref):
    px = posT_ref[0]
    py = posT_ref[1]
    pz = posT_ref[2]
    cidx1 = jax.lax.broadcasted_iota(jnp.int32, (B, C1), 1)
    s1, cx1, cy1, cz1 = _fps_level(px, py, pz, C1, cidx1)
    sel1_ref[...] = s1
    cen1T_ref[0] = cx1
    cen1T_ref[1] = cy1
    cen1T_ref[2] = cz1
    cidx2 = jax.lax.broadcasted_iota(jnp.int32, (B, C2), 1)
    s2, cx2, cy2, cz2 = _fps_level(cx1, cy1, cz1, C2, cidx2)
    sel2_ref[...] = s2
    cen2T_ref[0] = cx2
    cen2T_ref[1] = cy2
    cen2T_ref[2] = cz2


def _stage1_body(pos_ref, posT_ref, cen1_ref, w1_ref, b1_ref, w2a_ref, b2_ref,
                 x1_ref):
    pos2 = pos_ref[0]
    posT2 = posT_ref[0]
    cen4 = cen1_ref[0]
    cen3 = cen4[:, 0:3]
    w1 = w1_ref[...]
    a = jnp.dot(pos2, w1, preferred_element_type=jnp.float32)
    d = jnp.dot(cen3, w1, preferred_element_type=jnp.float32)
    pre = (a + b1_ref[...])[None, :, :] - d[:, None, :]
    h = jax.nn.relu(pre)

    dx = posT2[0:1, :] - cen4[:, 0:1]
    dy = posT2[1:2, :] - cen4[:, 1:2]
    dz = posT2[2:3, :] - cen4[:, 2:3]
    sq = dx * dx + dy * dy + dz * dz
    jidx = jax.lax.broadcasted_iota(jnp.int32, (C1, P), 1)
    mask = (sq < R1 * R1) & (jidx != cen4[:, 3:4].astype(jnp.int32))
    pen = jnp.where(mask, 0.0, NEG).astype(jnp.bfloat16)[:, :, None]
    msg = jnp.dot(h.reshape(C1 * P, 64).astype(jnp.bfloat16),
                  w2a_ref[...].astype(jnp.bfloat16),
                  preferred_element_type=jnp.float32).astype(jnp.bfloat16)
    msgm = msg.reshape(C1, P, 128) + pen
    m1 = jnp.max(msgm, axis=1)
    b2_16 = b2_ref[...].astype(jnp.bfloat16)
    x1_ref[0] = jnp.where(m1 > THRESH, m1 + b2_16,
                          jnp.bfloat16(0.0))


def _stage2_body(x1_ref, cen1_ref, cen1T_ref, cen2_ref, w21a_ref, w21b_ref,
                 b21_ref, w22a_ref, b22_ref, out_ref):
    x1 = x1_ref[0]
    cen4 = cen1_ref[0]
    cen1T2 = cen1T_ref[0]
    cen3 = cen4[:, 0:3]
    cen24 = cen2_ref[0]
    w21b = w21b_ref[...]
    u = (jnp.dot(x1, w21a_ref[...].astype(jnp.bfloat16),
                 preferred_element_type=jnp.float32)
         + jnp.dot(cen3, w21b, preferred_element_type=jnp.float32)
         + b21_ref[...])
    v = jnp.dot(cen24[:, 0:3], w21b, preferred_element_type=jnp.float32)
    u16 = u.astype(jnp.bfloat16)
    v16 = v.astype(jnp.bfloat16)
    pre2 = u16[None, :, :] - v16[:, None, :]
    h2 = jax.nn.relu(pre2)

    dx2 = cen1T2[0:1, :] - cen24[:, 0:1]
    dy2 = cen1T2[1:2, :] - cen24[:, 1:2]
    dz2 = cen1T2[2:3, :] - cen24[:, 2:3]
    sq2 = dx2 * dx2 + dy2 * dy2 + dz2 * dz2
    jidx2 = jax.lax.broadcasted_iota(jnp.int32, (C2, C1), 1)
    mask2 = (sq2 < R2 * R2) & (jidx2 != cen24[:, 3:4].astype(jnp.int32))
    pen2 = jnp.where(mask2, 0.0, NEG).astype(jnp.bfloat16)[:, :, None]
    msg2 = jnp.dot(h2.reshape(C2 * C1, 256), w22a_ref[...].astype(jnp.bfloat16),
                   preferred_element_type=jnp.float32).astype(jnp.bfloat16)
    msgm2 = msg2.reshape(C2, C1, 512) + pen2
    m2 = jnp.max(msgm2, axis=1).astype(jnp.float32)
    x2 = jnp.where(m2 > THRESH, m2 + b22_ref[...], 0.0)
    out_ref[0, 0] = jnp.max(x2, axis=0)


def kernel(pos, batch, sa1_W1, sa1_b1, sa1_W2, sa1_b2, sa2_W1, sa2_b1,
           sa2_W2, sa2_b2):
    del batch
    pos_b = pos.reshape(B, P, 3)
    posTb = jnp.transpose(pos_b, (0, 2, 1))

    sel1, cx1, cy1, cz1, sel2, cx2, cy2, cz2 = _fps_sc(posTb)
    cen1aug = jnp.stack([cx1, cy1, cz1, sel1], axis=-1)
    cen2aug = jnp.stack([cx2, cy2, cz2, sel2], axis=-1)

    x1 = pl.pallas_call(
        _stage1_body,
        grid=(B,),
        in_specs=[
            pl.BlockSpec((1, P, 3), lambda b: (b, 0, 0)),
            pl.BlockSpec((1, 3, P), lambda b: (b, 0, 0)),
            pl.BlockSpec((1, C1, 4), lambda b: (b, 0, 0)),
            pl.BlockSpec((3, 64), lambda b: (0, 0)),
            pl.BlockSpec((1, 64), lambda b: (0, 0)),
            pl.BlockSpec((64, 128), lambda b: (0, 0)),
            pl.BlockSpec((1, 128), lambda b: (0, 0)),
        ],
        out_specs=pl.BlockSpec((1, C1, 128), lambda b: (b, 0, 0)),
        out_shape=jax.ShapeDtypeStruct((B, C1, 128), jnp.bfloat16),
    )(pos_b, posTb, cen1aug, sa1_W1, sa1_b1.reshape(1, 64), sa1_W2,
      sa1_b2.reshape(1, 128))

    out = pl.pallas_call(
        _stage2_body,
        grid=(B,),
        in_specs=[
            pl.BlockSpec((1, C1, 128), lambda b: (b, 0, 0)),
            pl.BlockSpec((1, C1, 4), lambda b: (b, 0, 0)),
            pl.BlockSpec((1, 3, C1), lambda b: (b, 0, 0)),
            pl.BlockSpec((1, C2, 4), lambda b: (b, 0, 0)),
            pl.BlockSpec((128, 256), lambda b: (0, 0)),
            pl.BlockSpec((3, 256), lambda b: (0, 0)),
            pl.BlockSpec((1, 256), lambda b: (0, 0)),
            pl.BlockSpec((256, 512), lambda b: (0, 0)),
            pl.BlockSpec((1, 512), lambda b: (0, 0)),
        ],
        out_specs=pl.BlockSpec((1, 1, 512), lambda b: (b, 0, 0)),
        out_shape=jax.ShapeDtypeStruct((B, 1, 512), jnp.float32),
    )(x1, cen1aug, jnp.stack([cx1, cy1, cz1], axis=1), cen2aug,
      sa2_W1[:128], sa2_W1[128:], sa2_b1.reshape(1, 256), sa2_W2,
      sa2_b2.reshape(1, 512))

    return out.reshape(B, 512)
